# baseline (speedup 1.0000x reference)
import jax
import jax.numpy as jnp
from jax import lax
from jax.experimental import pallas as pl
from jax.experimental.pallas import tpu as pltpu
from jax.experimental.pallas import tpu_sc as plsc

_VOCAB = 1000000
_EMBED = 64
_OUT = 128
_BATCH = 4096
_SEQ = 200

_NC = 2
_NS = 16
_NW = _NC * _NS
_BPW = _BATCH // _NW
_IDXW = _BPW * _SEQ
_CH0 = 104
_CH1 = _SEQ - _CH0

_BCOLS = 7812
_CPW = 245
_PAIRS = _VOCAB // 2

_SC_PARAMS = pltpu.CompilerParams(use_tc_tiling_on_sc=True,
                                  needs_layout_passes=False)
_MESH = plsc.VectorSubcoreMesh(core_axis_name="c", subcore_axis_name="s",
                               num_cores=_NC, num_subcores=_NS)


def _bcast_lane(v, j):
    return lax.gather(
        v, jnp.full((16, 1), j, jnp.int32),
        dimension_numbers=lax.GatherDimensionNumbers(
            offset_dims=(), collapsed_slice_dims=(0,), start_index_map=(0,)),
        slice_sizes=(1,),
        mode=lax.GatherScatterMode.PROMISE_IN_BOUNDS)


_IOTA16 = lambda: lax.broadcasted_iota(jnp.int32, (16,), 0)


def _conv_body(tn_hbm, tail_hbm, out_hbm, stg0, stg1, po0, po1,
               isem0, isem1, osem0, osem1):
    wid = lax.axis_index("s") * _NC + lax.axis_index("c")

    def fire(t, stg, isem):
        b = jnp.minimum(wid + _NW * t, _BCOLS - 1)
        for a in range(8):
            pltpu.async_copy(
                tn_hbm.at[pl.ds(8 * a, 8), pl.ds(128 * b, 128)],
                stg.at[pl.ds(8 * a, 8), :], isem)

    def wait_in(stg, isem):
        pltpu.make_async_copy(
            tn_hbm.at[pl.ds(0, 64), pl.ds(0, 128)], stg, isem).wait()

    def repack(t, stg, po, osem):
        b = jnp.minimum(wid + _NW * t, _BCOLS - 1)

        def prow(p, _):
            for half in range(2):
                cj = jnp.full((16,), 2 * p + half, jnp.int32)
                for q in range(4):
                    v = plsc.load_gather(stg, [_IOTA16() + 16 * q, cj])
                    po[p, pl.ds(64 * half + 16 * q, 16)] = v
            return 0

        lax.fori_loop(0, 64, prow, 0)
        pltpu.async_copy(po, out_hbm.at[pl.ds(64 * b, 64), :], osem)

    def wait_out(po, osem):
        pltpu.make_async_copy(
            tn_hbm.at[pl.ds(0, 64), pl.ds(0, 128)], po, osem).wait()

    fire(0, stg0, isem0)
    fire(1, stg1, isem1)

    def step(k, _):
        t0 = 2 * k
        wait_in(stg0, isem0)

        @pl.when(k > 0)
        def _():
            wait_out(po0, osem0)

        repack(t0, stg0, po0, osem0)

        @pl.when(t0 + 2 < _CPW)
        def _():
            fire(t0 + 2, stg0, isem0)

        wait_in(stg1, isem1)

        @pl.when(k > 0)
        def _():
            wait_out(po1, osem1)

        repack(t0 + 1, stg1, po1, osem1)

        @pl.when(t0 + 3 < _CPW)
        def _():
            fire(t0 + 3, stg1, isem1)

        return 0

    lax.fori_loop(0, _CPW // 2, step, 0)

    t_last = _CPW - 1
    wait_in(stg0, isem0)
    wait_out(po0, osem0)
    repack(t_last, stg0, po0, osem0)
    wait_out(po0, osem0)
    wait_out(po1, osem1)

    @pl.when(wid == 0)
    def _():
        pltpu.sync_copy(tail_hbm, out_hbm.at[pl.ds(_PAIRS - 32, 32)])


_conv = pl.kernel(
    _conv_body,
    out_type=jax.ShapeDtypeStruct((_PAIRS, 2 * _EMBED), jnp.float32),
    mesh=_MESH,
    compiler_params=_SC_PARAMS,
    scratch_types=[
        pltpu.VMEM((_EMBED, 128), jnp.float32),
        pltpu.VMEM((_EMBED, 128), jnp.float32),
        pltpu.VMEM((_EMBED, 128), jnp.float32),
        pltpu.VMEM((_EMBED, 128), jnp.float32),
        pltpu.SemaphoreType.DMA,
        pltpu.SemaphoreType.DMA,
        pltpu.SemaphoreType.DMA,
        pltpu.SemaphoreType.DMA,
    ],
)


def _pool_body(x_hbm, table_hbm, out_hbm, idx_v, kidx0, kidx1, rows0, rows1,
               out_v, sem0, sem1):
    wid = lax.axis_index("s") * _NC + lax.axis_index("c")
    idx_base = wid * _IDXW

    pltpu.sync_copy(x_hbm.at[pl.ds(idx_base, _IDXW)],
                    idx_v.at[pl.ds(0, _IDXW)])

    def fire(e, kidx, rows_ref, sem):
        off = e * _SEQ
        for m in range(12):
            kidx[pl.ds(16 * m, 16)] = lax.shift_right_logical(
                idx_v[pl.ds(off + 16 * m, 16)], 1)
        kidx[pl.ds(_SEQ - 16, 16)] = lax.shift_right_logical(
            idx_v[pl.ds(off + _SEQ - 16, 16)], 1)
        pltpu.async_copy(
            table_hbm.at[kidx.at[pl.ds(0, _CH0)]],
            rows_ref.at[pl.ds(0, _CH0)], sem)
        pltpu.async_copy(
            table_hbm.at[kidx.at[pl.ds(_CH0, _CH1)]],
            rows_ref.at[pl.ds(_CH0, _CH1)], sem)

    def wait(rows_ref, sem):
        pltpu.make_async_copy(
            table_hbm.at[pl.ds(0, _SEQ)], rows_ref, sem).wait()

    def accum(rows_ref, e):
        off = e * _SEQ

        def rows16(g, carry, nrows):
            hv = (idx_v[pl.ds(off + 16 * g, 16)] & 1) * 64
            for j in range(nrows):
                sj = jnp.full((16,), 16 * g + j, jnp.int32)
                base = _bcast_lane(hv, j) + _IOTA16()
                nxt = []
                for m in range(4):
                    v = plsc.load_gather(rows_ref, [sj, base + 16 * m])
                    nxt.append(carry[m] + v)
                carry = tuple(nxt)
            return carry

        z = jnp.zeros((16,), jnp.float32)
        a0, a1, a2, a3 = lax.fori_loop(
            0, _SEQ // 16, lambda g, c: rows16(g, c, 16), (z, z, z, z))
        a0, a1, a2, a3 = rows16(_SEQ // 16, (a0, a1, a2, a3), _SEQ % 16)
        out_v[e, pl.ds(0, 16)] = a0
        out_v[e, pl.ds(16, 16)] = a1
        out_v[e, pl.ds(32, 16)] = a2
        out_v[e, pl.ds(48, 16)] = a3

    fire(0, kidx0, rows0, sem0)
    fire(1, kidx1, rows1, sem1)

    def step(k, _):
        e0 = 2 * k
        wait(rows0, sem0)
        accum(rows0, e0)

        @pl.when(k < _BPW // 2 - 1)
        def _():
            fire(e0 + 2, kidx0, rows0, sem0)

        wait(rows1, sem1)
        accum(rows1, e0 + 1)

        @pl.when(k < _BPW // 2 - 1)
        def _():
            fire(e0 + 3, kidx1, rows1, sem1)

        return 0

    lax.fori_loop(0, _BPW // 2, step, 0)

    pltpu.sync_copy(out_v, out_hbm.at[pl.ds(wid * _BPW, _BPW)])


_pool = pl.kernel(
    _pool_body,
    out_type=jax.ShapeDtypeStruct((_BATCH, _EMBED), jnp.float32),
    mesh=_MESH,
    compiler_params=_SC_PARAMS,
    scratch_types=[
        pltpu.VMEM((_IDXW + 16,), jnp.int32),
        pltpu.VMEM((_SEQ,), jnp.int32),
        pltpu.VMEM((_SEQ,), jnp.int32),
        pltpu.VMEM((_SEQ, 2 * _EMBED), jnp.float32),
        pltpu.VMEM((_SEQ, 2 * _EMBED), jnp.float32),
        pltpu.VMEM((_BPW, _EMBED), jnp.float32),
        pltpu.SemaphoreType.DMA,
        pltpu.SemaphoreType.DMA,
    ],
)


def _mlp_body(p_ref, w_ref, b_ref, o_ref):
    pooled = p_ref[...] * (1.0 / _SEQ)
    o_ref[...] = (
        jnp.dot(pooled, w_ref[...], preferred_element_type=jnp.float32)
        + b_ref[...])


_mlp = pl.pallas_call(
    _mlp_body,
    out_shape=jax.ShapeDtypeStruct((_BATCH, _OUT), jnp.float32),
)


@jax.jit
def kernel(x, table, W1, b1):
    x_flat = x.reshape(-1).astype(jnp.int32)
    tail_pairs = table[_VOCAB - 64:].reshape(32, 2 * _EMBED)
    pairs = _conv(table.T, tail_pairs)
    sums = _pool(x_flat, pairs)
    return _mlp(sums, W1, b1.reshape(1, _OUT))


# speedup vs baseline: 1.8016x; 1.8016x over previous
import jax
import jax.numpy as jnp
from jax import lax
from jax.experimental import pallas as pl
from jax.experimental.pallas import tpu as pltpu
from jax.experimental.pallas import tpu_sc as plsc

_VOCAB = 1000000
_EMBED = 64
_OUT = 128
_BATCH = 4096
_SEQ = 200

_NC = 2
_NS = 16
_NW = _NC * _NS
_BPW = _BATCH // _NW
_IDXW = _BPW * _SEQ
_CH0 = 104
_CH1 = _SEQ - _CH0

_BCOLS = 7812
_CPW = 245
_PAIRS = _VOCAB // 2

_SC_PARAMS = pltpu.CompilerParams(use_tc_tiling_on_sc=True,
                                  needs_layout_passes=False)
_MESH = plsc.VectorSubcoreMesh(core_axis_name="c", subcore_axis_name="s",
                               num_cores=_NC, num_subcores=_NS)


def _bcast_lane(v, j):
    return lax.gather(
        v, jnp.full((16, 1), j, jnp.int32),
        dimension_numbers=lax.GatherDimensionNumbers(
            offset_dims=(), collapsed_slice_dims=(0,), start_index_map=(0,)),
        slice_sizes=(1,),
        mode=lax.GatherScatterMode.PROMISE_IN_BOUNDS)


_IOTA16 = lambda: lax.broadcasted_iota(jnp.int32, (16,), 0)


def _conv_body(tn_hbm, tail_hbm, out_hbm, stg0, stg1, po0, po1,
               isem0, isem1, osem0, osem1):
    wid = lax.axis_index("s") * _NC + lax.axis_index("c")

    def fire(t, stg, isem):
        b = jnp.minimum(wid + _NW * t, _BCOLS - 1)
        for a in range(8):
            pltpu.async_copy(
                tn_hbm.at[pl.ds(8 * a, 8), pl.ds(128 * b, 128)],
                stg.at[pl.ds(8 * a, 8), :], isem)

    def wait_in(stg, isem):
        pltpu.make_async_copy(
            tn_hbm.at[pl.ds(0, 64), pl.ds(0, 128)], stg, isem).wait()

    def repack(t, stg, po, osem):
        b = jnp.minimum(wid + _NW * t, _BCOLS - 1)

        @plsc.parallel_loop(0, 64, unroll=8)
        def prow(p):
            for half in range(2):
                cj = jnp.full((16,), 2 * p + half, jnp.int32)
                for q in range(4):
                    v = plsc.load_gather(stg, [_IOTA16() + 16 * q, cj])
                    po[p, pl.ds(64 * half + 16 * q, 16)] = v
        pltpu.async_copy(po, out_hbm.at[pl.ds(64 * b, 64), :], osem)

    def wait_out(po, osem):
        pltpu.make_async_copy(
            tn_hbm.at[pl.ds(0, 64), pl.ds(0, 128)], po, osem).wait()

    fire(0, stg0, isem0)
    fire(1, stg1, isem1)

    def step(k, _):
        t0 = 2 * k
        wait_in(stg0, isem0)

        @pl.when(k > 0)
        def _():
            wait_out(po0, osem0)

        repack(t0, stg0, po0, osem0)

        @pl.when(t0 + 2 < _CPW)
        def _():
            fire(t0 + 2, stg0, isem0)

        wait_in(stg1, isem1)

        @pl.when(k > 0)
        def _():
            wait_out(po1, osem1)

        repack(t0 + 1, stg1, po1, osem1)

        @pl.when(t0 + 3 < _CPW)
        def _():
            fire(t0 + 3, stg1, isem1)

        return 0

    lax.fori_loop(0, _CPW // 2, step, 0)

    t_last = _CPW - 1
    wait_in(stg0, isem0)
    wait_out(po0, osem0)
    repack(t_last, stg0, po0, osem0)
    wait_out(po0, osem0)
    wait_out(po1, osem1)

    @pl.when(wid == 0)
    def _():
        pltpu.sync_copy(tail_hbm, out_hbm.at[pl.ds(_PAIRS - 32, 32)])


_conv = pl.kernel(
    _conv_body,
    out_type=jax.ShapeDtypeStruct((_PAIRS, 2 * _EMBED), jnp.float32),
    mesh=_MESH,
    compiler_params=_SC_PARAMS,
    scratch_types=[
        pltpu.VMEM((_EMBED, 128), jnp.float32),
        pltpu.VMEM((_EMBED, 128), jnp.float32),
        pltpu.VMEM((_EMBED, 128), jnp.float32),
        pltpu.VMEM((_EMBED, 128), jnp.float32),
        pltpu.SemaphoreType.DMA,
        pltpu.SemaphoreType.DMA,
        pltpu.SemaphoreType.DMA,
        pltpu.SemaphoreType.DMA,
    ],
)


def _pool_body(x_hbm, table_hbm, out_hbm, idx_v, kidx0, kidx1, rows0, rows1,
               out_v, sem0, sem1):
    wid = lax.axis_index("s") * _NC + lax.axis_index("c")
    idx_base = wid * _IDXW

    pltpu.sync_copy(x_hbm.at[pl.ds(idx_base, _IDXW)],
                    idx_v.at[pl.ds(0, _IDXW)])

    def fire(e, kidx, rows_ref, sem):
        off = e * _SEQ
        for m in range(12):
            kidx[pl.ds(16 * m, 16)] = lax.shift_right_logical(
                idx_v[pl.ds(off + 16 * m, 16)], 1)
        kidx[pl.ds(_SEQ - 16, 16)] = lax.shift_right_logical(
            idx_v[pl.ds(off + _SEQ - 16, 16)], 1)
        pltpu.async_copy(
            table_hbm.at[kidx.at[pl.ds(0, _CH0)]],
            rows_ref.at[pl.ds(0, _CH0)], sem)
        pltpu.async_copy(
            table_hbm.at[kidx.at[pl.ds(_CH0, _CH1)]],
            rows_ref.at[pl.ds(_CH0, _CH1)], sem)

    def wait(rows_ref, sem):
        pltpu.make_async_copy(
            table_hbm.at[pl.ds(0, _SEQ)], rows_ref, sem).wait()

    def accum(rows_ref, e):
        off = e * _SEQ

        def rows16(g, carry, nrows):
            hv = (idx_v[pl.ds(off + 16 * g, 16)] & 1) * 64
            for j in range(nrows):
                s = 16 * g + j
                h = hv[j]
                nxt = []
                for m in range(4):
                    nxt.append(carry[m] + rows_ref[s, pl.ds(h + 16 * m, 16)])
                carry = tuple(nxt)
            return carry

        z = jnp.zeros((16,), jnp.float32)
        a0, a1, a2, a3 = lax.fori_loop(
            0, _SEQ // 16, lambda g, c: rows16(g, c, 16), (z, z, z, z))
        a0, a1, a2, a3 = rows16(_SEQ // 16, (a0, a1, a2, a3), _SEQ % 16)
        out_v[e, pl.ds(0, 16)] = a0
        out_v[e, pl.ds(16, 16)] = a1
        out_v[e, pl.ds(32, 16)] = a2
        out_v[e, pl.ds(48, 16)] = a3

    fire(0, kidx0, rows0, sem0)
    fire(1, kidx1, rows1, sem1)

    def step(k, _):
        e0 = 2 * k
        wait(rows0, sem0)
        accum(rows0, e0)

        @pl.when(k < _BPW // 2 - 1)
        def _():
            fire(e0 + 2, kidx0, rows0, sem0)

        wait(rows1, sem1)
        accum(rows1, e0 + 1)

        @pl.when(k < _BPW // 2 - 1)
        def _():
            fire(e0 + 3, kidx1, rows1, sem1)

        return 0

    lax.fori_loop(0, _BPW // 2, step, 0)

    pltpu.sync_copy(out_v, out_hbm.at[pl.ds(wid * _BPW, _BPW)])


_pool = pl.kernel(
    _pool_body,
    out_type=jax.ShapeDtypeStruct((_BATCH, _EMBED), jnp.float32),
    mesh=_MESH,
    compiler_params=_SC_PARAMS,
    scratch_types=[
        pltpu.VMEM((_IDXW + 16,), jnp.int32),
        pltpu.VMEM((_SEQ,), jnp.int32),
        pltpu.VMEM((_SEQ,), jnp.int32),
        pltpu.VMEM((_SEQ, 2 * _EMBED), jnp.float32),
        pltpu.VMEM((_SEQ, 2 * _EMBED), jnp.float32),
        pltpu.VMEM((_BPW, _EMBED), jnp.float32),
        pltpu.SemaphoreType.DMA,
        pltpu.SemaphoreType.DMA,
    ],
)


def _mlp_body(p_ref, w_ref, b_ref, o_ref):
    pooled = p_ref[...] * (1.0 / _SEQ)
    o_ref[...] = (
        jnp.dot(pooled, w_ref[...], preferred_element_type=jnp.float32)
        + b_ref[...])


_mlp = pl.pallas_call(
    _mlp_body,
    out_shape=jax.ShapeDtypeStruct((_BATCH, _OUT), jnp.float32),
)


@jax.jit
def kernel(x, table, W1, b1):
    x_flat = x.reshape(-1).astype(jnp.int32)
    tail_pairs = table[_VOCAB - 64:].reshape(32, 2 * _EMBED)
    pairs = _conv(table.T, tail_pairs)
    sums = _pool(x_flat, pairs)
    return _mlp(sums, W1, b1.reshape(1, _OUT))


# speedup vs baseline: 1.8061x; 1.0025x over previous
import jax
import jax.numpy as jnp
from jax import lax
from jax.experimental import pallas as pl
from jax.experimental.pallas import tpu as pltpu
from jax.experimental.pallas import tpu_sc as plsc

_VOCAB = 1000000
_EMBED = 64
_OUT = 128
_BATCH = 4096
_SEQ = 200

_NC = 2
_NS = 16
_NW = _NC * _NS
_BPW = _BATCH // _NW
_IDXW = _BPW * _SEQ
_CH0 = 104
_CH1 = _SEQ - _CH0

_BCOLS = 7812
_CPW = 245
_PAIRS = _VOCAB // 2

_SC_PARAMS = pltpu.CompilerParams(use_tc_tiling_on_sc=True,
                                  needs_layout_passes=False)
_MESH = plsc.VectorSubcoreMesh(core_axis_name="c", subcore_axis_name="s",
                               num_cores=_NC, num_subcores=_NS)


def _bcast_lane(v, j):
    return lax.gather(
        v, jnp.full((16, 1), j, jnp.int32),
        dimension_numbers=lax.GatherDimensionNumbers(
            offset_dims=(), collapsed_slice_dims=(0,), start_index_map=(0,)),
        slice_sizes=(1,),
        mode=lax.GatherScatterMode.PROMISE_IN_BOUNDS)


_IOTA16 = lambda: lax.broadcasted_iota(jnp.int32, (16,), 0)


def _conv_body(tn_hbm, tail_hbm, out_hbm, stg0, stg1, po0, po1,
               isem0, isem1, osem0, osem1):
    wid = lax.axis_index("s") * _NC + lax.axis_index("c")

    def fire(t, stg, isem):
        b = jnp.minimum(wid + _NW * t, _BCOLS - 1)
        for a in range(8):
            pltpu.async_copy(
                tn_hbm.at[pl.ds(8 * a, 8), pl.ds(128 * b, 128)],
                stg.at[pl.ds(8 * a, 8), pl.ds(0, 128)], isem)

    def wait_in(stg, isem):
        pltpu.make_async_copy(
            tn_hbm.at[pl.ds(0, 64), pl.ds(0, 128)],
            stg.at[pl.ds(0, 64), pl.ds(0, 128)], isem).wait()

    def repack(t, stg, po, osem):
        b = jnp.minimum(wid + _NW * t, _BCOLS - 1)

        @plsc.parallel_loop(0, 64, unroll=8)
        def prow(p):
            for half in range(2):
                cj = jnp.full((16,), 2 * p + half, jnp.int32)
                for q in range(4):
                    v = plsc.load_gather(stg, [_IOTA16() + 16 * q, cj])
                    po[p, pl.ds(64 * half + 16 * q, 16)] = v
        pltpu.async_copy(po, out_hbm.at[pl.ds(64 * b, 64), :], osem)

    def wait_out(po, osem):
        pltpu.make_async_copy(
            tn_hbm.at[pl.ds(0, 64), pl.ds(0, 128)], po, osem).wait()

    fire(0, stg0, isem0)
    fire(1, stg1, isem1)

    def step(k, _):
        t0 = 2 * k
        wait_in(stg0, isem0)

        @pl.when(k > 0)
        def _():
            wait_out(po0, osem0)

        repack(t0, stg0, po0, osem0)

        @pl.when(t0 + 2 < _CPW)
        def _():
            fire(t0 + 2, stg0, isem0)

        wait_in(stg1, isem1)

        @pl.when(k > 0)
        def _():
            wait_out(po1, osem1)

        repack(t0 + 1, stg1, po1, osem1)

        @pl.when(t0 + 3 < _CPW)
        def _():
            fire(t0 + 3, stg1, isem1)

        return 0

    lax.fori_loop(0, _CPW // 2, step, 0)

    t_last = _CPW - 1
    wait_in(stg0, isem0)
    wait_out(po0, osem0)
    repack(t_last, stg0, po0, osem0)
    wait_out(po0, osem0)
    wait_out(po1, osem1)

    @pl.when(wid == 0)
    def _():
        pltpu.sync_copy(tail_hbm, out_hbm.at[pl.ds(_PAIRS - 32, 32)])


_conv = pl.kernel(
    _conv_body,
    out_type=jax.ShapeDtypeStruct((_PAIRS, 2 * _EMBED), jnp.float32),
    mesh=_MESH,
    compiler_params=_SC_PARAMS,
    scratch_types=[
        pltpu.VMEM((_EMBED, 129), jnp.float32),
        pltpu.VMEM((_EMBED, 129), jnp.float32),
        pltpu.VMEM((_EMBED, 128), jnp.float32),
        pltpu.VMEM((_EMBED, 128), jnp.float32),
        pltpu.SemaphoreType.DMA,
        pltpu.SemaphoreType.DMA,
        pltpu.SemaphoreType.DMA,
        pltpu.SemaphoreType.DMA,
    ],
)


def _pool_body(x_hbm, table_hbm, out_hbm, idx_v, kidx0, kidx1, rows0, rows1,
               out_v, sem0, sem1):
    wid = lax.axis_index("s") * _NC + lax.axis_index("c")
    idx_base = wid * _IDXW

    pltpu.sync_copy(x_hbm.at[pl.ds(idx_base, _IDXW)],
                    idx_v.at[pl.ds(0, _IDXW)])

    def fire(e, kidx, rows_ref, sem):
        off = e * _SEQ
        for m in range(12):
            kidx[pl.ds(16 * m, 16)] = lax.shift_right_logical(
                idx_v[pl.ds(off + 16 * m, 16)], 1)
        kidx[pl.ds(_SEQ - 16, 16)] = lax.shift_right_logical(
            idx_v[pl.ds(off + _SEQ - 16, 16)], 1)
        pltpu.async_copy(
            table_hbm.at[kidx.at[pl.ds(0, _CH0)]],
            rows_ref.at[pl.ds(0, _CH0)], sem)
        pltpu.async_copy(
            table_hbm.at[kidx.at[pl.ds(_CH0, _CH1)]],
            rows_ref.at[pl.ds(_CH0, _CH1)], sem)

    def wait(rows_ref, sem):
        pltpu.make_async_copy(
            table_hbm.at[pl.ds(0, _SEQ)], rows_ref, sem).wait()

    def accum(rows_ref, e):
        off = e * _SEQ

        def rows16(g, carry, nrows):
            hv = (idx_v[pl.ds(off + 16 * g, 16)] & 1) * 64
            for j in range(nrows):
                s = 16 * g + j
                h = hv[j]
                nxt = []
                for m in range(4):
                    nxt.append(carry[m] + rows_ref[s, pl.ds(h + 16 * m, 16)])
                carry = tuple(nxt)
            return carry

        z = jnp.zeros((16,), jnp.float32)
        a0, a1, a2, a3 = lax.fori_loop(
            0, _SEQ // 16, lambda g, c: rows16(g, c, 16), (z, z, z, z))
        a0, a1, a2, a3 = rows16(_SEQ // 16, (a0, a1, a2, a3), _SEQ % 16)
        out_v[e, pl.ds(0, 16)] = a0
        out_v[e, pl.ds(16, 16)] = a1
        out_v[e, pl.ds(32, 16)] = a2
        out_v[e, pl.ds(48, 16)] = a3

    fire(0, kidx0, rows0, sem0)
    fire(1, kidx1, rows1, sem1)

    def step(k, _):
        e0 = 2 * k
        wait(rows0, sem0)
        accum(rows0, e0)

        @pl.when(k < _BPW // 2 - 1)
        def _():
            fire(e0 + 2, kidx0, rows0, sem0)

        wait(rows1, sem1)
        accum(rows1, e0 + 1)

        @pl.when(k < _BPW // 2 - 1)
        def _():
            fire(e0 + 3, kidx1, rows1, sem1)

        return 0

    lax.fori_loop(0, _BPW // 2, step, 0)

    pltpu.sync_copy(out_v, out_hbm.at[pl.ds(wid * _BPW, _BPW)])


_pool = pl.kernel(
    _pool_body,
    out_type=jax.ShapeDtypeStruct((_BATCH, _EMBED), jnp.float32),
    mesh=_MESH,
    compiler_params=_SC_PARAMS,
    scratch_types=[
        pltpu.VMEM((_IDXW + 16,), jnp.int32),
        pltpu.VMEM((_SEQ,), jnp.int32),
        pltpu.VMEM((_SEQ,), jnp.int32),
        pltpu.VMEM((_SEQ, 2 * _EMBED), jnp.float32),
        pltpu.VMEM((_SEQ, 2 * _EMBED), jnp.float32),
        pltpu.VMEM((_BPW, _EMBED), jnp.float32),
        pltpu.SemaphoreType.DMA,
        pltpu.SemaphoreType.DMA,
    ],
)


def _mlp_body(p_ref, w_ref, b_ref, o_ref):
    pooled = p_ref[...] * (1.0 / _SEQ)
    o_ref[...] = (
        jnp.dot(pooled, w_ref[...], preferred_element_type=jnp.float32)
        + b_ref[...])


_mlp = pl.pallas_call(
    _mlp_body,
    out_shape=jax.ShapeDtypeStruct((_BATCH, _OUT), jnp.float32),
)


@jax.jit
def kernel(x, table, W1, b1):
    x_flat = x.reshape(-1).astype(jnp.int32)
    tail_pairs = table[_VOCAB - 64:].reshape(32, 2 * _EMBED)
    pairs = _conv(table.T, tail_pairs)
    sums = _pool(x_flat, pairs)
    return _mlp(sums, W1, b1.reshape(1, _OUT))


# speedup vs baseline: 2.7282x; 1.5105x over previous
import jax
import jax.numpy as jnp
from jax import lax
from jax.experimental import pallas as pl
from jax.experimental.pallas import tpu as pltpu
from jax.experimental.pallas import tpu_sc as plsc

_VOCAB = 1000000
_EMBED = 64
_OUT = 128
_BATCH = 4096
_SEQ = 200

_NC = 2
_NS = 16
_NW = _NC * _NS
_BPW = _BATCH // _NW
_IDXW = _BPW * _SEQ
_CH0 = 104
_CH1 = _SEQ - _CH0

_BCOLS = 7812
_CPW = 245
_PAIRS = _VOCAB // 2

_SC_PARAMS = pltpu.CompilerParams(use_tc_tiling_on_sc=True,
                                  needs_layout_passes=False)
_MESH = plsc.VectorSubcoreMesh(core_axis_name="c", subcore_axis_name="s",
                               num_cores=_NC, num_subcores=_NS)


def _bcast_lane(v, j):
    return lax.gather(
        v, jnp.full((16, 1), j, jnp.int32),
        dimension_numbers=lax.GatherDimensionNumbers(
            offset_dims=(), collapsed_slice_dims=(0,), start_index_map=(0,)),
        slice_sizes=(1,),
        mode=lax.GatherScatterMode.PROMISE_IN_BOUNDS)


_IOTA16 = lambda: lax.broadcasted_iota(jnp.int32, (16,), 0)


_TCHUNK = 2048
_TGRID = (_VOCAB + _TCHUNK - 1) // _TCHUNK
_PROWS = _TGRID * (_TCHUNK // 2)


def _tconv_body(t_ref, o_ref):
    t = t_ref[...]
    left = lax.transpose(t[:, : _TCHUNK // 2], (1, 0))
    right = lax.transpose(t[:, _TCHUNK // 2:], (1, 0))
    o_ref[...] = lax.concatenate([left, right], 1)


_tconv = pl.pallas_call(
    _tconv_body,
    grid=(_TGRID,),
    in_specs=[pl.BlockSpec((_EMBED, _TCHUNK), lambda i: (0, i))],
    out_specs=pl.BlockSpec((_TCHUNK // 2, 2 * _EMBED), lambda i: (i, 0)),
    out_shape=jax.ShapeDtypeStruct((_PROWS, 2 * _EMBED), jnp.float32),
)


def _pool_body(x_hbm, table_hbm, out_hbm, idx_v, kidx0, kidx1, rows0, rows1,
               out_v, sem0, sem1):
    wid = lax.axis_index("s") * _NC + lax.axis_index("c")
    idx_base = wid * _IDXW

    pltpu.sync_copy(x_hbm.at[pl.ds(idx_base, _IDXW)],
                    idx_v.at[pl.ds(0, _IDXW)])

    def fire(e, kidx, rows_ref, sem):
        off = e * _SEQ
        def pair_id(v):
            return lax.shift_right_logical(v & ~2047, 1) | (v & 1023)

        for m in range(12):
            kidx[pl.ds(16 * m, 16)] = pair_id(idx_v[pl.ds(off + 16 * m, 16)])
        kidx[pl.ds(_SEQ - 16, 16)] = pair_id(
            idx_v[pl.ds(off + _SEQ - 16, 16)])
        pltpu.async_copy(
            table_hbm.at[kidx.at[pl.ds(0, _CH0)]],
            rows_ref.at[pl.ds(0, _CH0)], sem)
        pltpu.async_copy(
            table_hbm.at[kidx.at[pl.ds(_CH0, _CH1)]],
            rows_ref.at[pl.ds(_CH0, _CH1)], sem)

    def wait(rows_ref, sem):
        pltpu.make_async_copy(
            table_hbm.at[pl.ds(0, _SEQ)], rows_ref, sem).wait()

    def accum(rows_ref, e):
        off = e * _SEQ

        def rows16(g, carry, nrows):
            hv = (lax.shift_right_logical(
                idx_v[pl.ds(off + 16 * g, 16)], 10) & 1) * 64
            for j in range(nrows):
                s = 16 * g + j
                h = hv[j]
                nxt = []
                for m in range(4):
                    nxt.append(carry[m] + rows_ref[s, pl.ds(h + 16 * m, 16)])
                carry = tuple(nxt)
            return carry

        z = jnp.zeros((16,), jnp.float32)
        a0, a1, a2, a3 = lax.fori_loop(
            0, _SEQ // 16, lambda g, c: rows16(g, c, 16), (z, z, z, z))
        a0, a1, a2, a3 = rows16(_SEQ // 16, (a0, a1, a2, a3), _SEQ % 16)
        out_v[e, pl.ds(0, 16)] = a0
        out_v[e, pl.ds(16, 16)] = a1
        out_v[e, pl.ds(32, 16)] = a2
        out_v[e, pl.ds(48, 16)] = a3

    fire(0, kidx0, rows0, sem0)
    fire(1, kidx1, rows1, sem1)

    def step(k, _):
        e0 = 2 * k
        wait(rows0, sem0)
        accum(rows0, e0)

        @pl.when(k < _BPW // 2 - 1)
        def _():
            fire(e0 + 2, kidx0, rows0, sem0)

        wait(rows1, sem1)
        accum(rows1, e0 + 1)

        @pl.when(k < _BPW // 2 - 1)
        def _():
            fire(e0 + 3, kidx1, rows1, sem1)

        return 0

    lax.fori_loop(0, _BPW // 2, step, 0)

    pltpu.sync_copy(out_v, out_hbm.at[pl.ds(wid * _BPW, _BPW)])


_pool = pl.kernel(
    _pool_body,
    out_type=jax.ShapeDtypeStruct((_BATCH, _EMBED), jnp.float32),
    mesh=_MESH,
    compiler_params=_SC_PARAMS,
    scratch_types=[
        pltpu.VMEM((_IDXW + 16,), jnp.int32),
        pltpu.VMEM((_SEQ,), jnp.int32),
        pltpu.VMEM((_SEQ,), jnp.int32),
        pltpu.VMEM((_SEQ, 2 * _EMBED), jnp.float32),
        pltpu.VMEM((_SEQ, 2 * _EMBED), jnp.float32),
        pltpu.VMEM((_BPW, _EMBED), jnp.float32),
        pltpu.SemaphoreType.DMA,
        pltpu.SemaphoreType.DMA,
    ],
)


def _mlp_body(p_ref, w_ref, b_ref, o_ref):
    pooled = p_ref[...] * (1.0 / _SEQ)
    o_ref[...] = (
        jnp.dot(pooled, w_ref[...], preferred_element_type=jnp.float32)
        + b_ref[...])


_mlp = pl.pallas_call(
    _mlp_body,
    out_shape=jax.ShapeDtypeStruct((_BATCH, _OUT), jnp.float32),
)


@jax.jit
def kernel(x, table, W1, b1):
    x_flat = x.reshape(-1).astype(jnp.int32)
    pairs = _tconv(table.T)
    sums = _pool(x_flat, pairs)
    return _mlp(sums, W1, b1.reshape(1, _OUT))


# speedup vs baseline: 4.4916x; 1.6463x over previous
import jax
import jax.numpy as jnp
from jax import lax
from jax.experimental import pallas as pl
from jax.experimental.pallas import tpu as pltpu
from jax.experimental.pallas import tpu_sc as plsc

_VOCAB = 1000000
_EMBED = 64
_OUT = 128
_BATCH = 4096
_SEQ = 200

_NC = 2
_NS = 16
_NW = _NC * _NS
_BPW = _BATCH // _NW
_IDXW = _BPW * _SEQ
_CH0 = 104
_CH1 = _SEQ - _CH0

_SC_PARAMS = pltpu.CompilerParams(use_tc_tiling_on_sc=True,
                                  needs_layout_passes=False)
_MESH = plsc.VectorSubcoreMesh(core_axis_name="c", subcore_axis_name="s",
                               num_cores=_NC, num_subcores=_NS)


_TCHUNK = 32768
_TGRID = (_VOCAB + _TCHUNK - 1) // _TCHUNK
_HSHIFT = (_TCHUNK // 2).bit_length() - 1
_PROWS = _TGRID * (_TCHUNK // 2)


def _tconv_body(t_ref, o_ref):
    t = t_ref[...]
    left = lax.transpose(t[:, : _TCHUNK // 2], (1, 0))
    right = lax.transpose(t[:, _TCHUNK // 2:], (1, 0))
    o_ref[...] = lax.concatenate([left, right], 1)


_tconv = pl.pallas_call(
    _tconv_body,
    grid=(_TGRID,),
    in_specs=[pl.BlockSpec((_EMBED, _TCHUNK), lambda i: (0, i))],
    out_specs=pl.BlockSpec((_TCHUNK // 2, 2 * _EMBED), lambda i: (i, 0)),
    out_shape=jax.ShapeDtypeStruct((_PROWS, 2 * _EMBED), jnp.float32),
)


def _pool_body(x_hbm, table_hbm, out_hbm, idx_v, kidx0, kidx1, kidx2,
               rows0, rows1, rows2, out_v, sem0a, sem0b, sem1a, sem1b,
               sem2a, sem2b):
    wid = lax.axis_index("s") * _NC + lax.axis_index("c")
    idx_base = wid * _IDXW

    pltpu.sync_copy(x_hbm.at[pl.ds(idx_base, _IDXW)],
                    idx_v.at[pl.ds(0, _IDXW)])

    def fire(e, kidx, rows_ref, sem):
        off = e * _SEQ
        def pair_id(v):
            return (lax.shift_right_logical(v & ~(_TCHUNK - 1), 1)
                    | (v & (_TCHUNK // 2 - 1)))

        for m in range(12):
            kidx[pl.ds(16 * m, 16)] = pair_id(idx_v[pl.ds(off + 16 * m, 16)])
        kidx[pl.ds(_SEQ - 16, 16)] = pair_id(
            idx_v[pl.ds(off + _SEQ - 16, 16)])
        sem_a, sem_b = sem
        pltpu.async_copy(
            table_hbm.at[kidx.at[pl.ds(0, _CH0)]],
            rows_ref.at[pl.ds(0, _CH0)], sem_a)
        pltpu.async_copy(
            table_hbm.at[kidx.at[pl.ds(_CH0, _CH1)]],
            rows_ref.at[pl.ds(_CH0, _CH1)], sem_b)

    def wait_a(rows_ref, sem):
        pltpu.make_async_copy(
            table_hbm.at[pl.ds(0, _CH0)],
            rows_ref.at[pl.ds(0, _CH0)], sem[0]).wait()

    def wait_b(rows_ref, sem):
        pltpu.make_async_copy(
            table_hbm.at[pl.ds(0, _CH1)],
            rows_ref.at[pl.ds(_CH0, _CH1)], sem[1]).wait()

    def rows16(rows_ref, off, start, carry, nrows):
        hv = (lax.shift_right_logical(
            idx_v[pl.ds(off + start, 16)], _HSHIFT) & 1) * 64
        for j in range(nrows):
            s = start + j
            h = hv[j]
            nxt = []
            for m in range(4):
                nxt.append(carry[m] + rows_ref[s, pl.ds(h + 16 * m, 16)])
            carry = tuple(nxt)
        return carry

    def accum_a(rows_ref, e):
        off = e * _SEQ
        z = jnp.zeros((16,), jnp.float32)
        c = lax.fori_loop(
            0, _CH0 // 16,
            lambda g, c: rows16(rows_ref, off, 16 * g, c, 16), (z, z, z, z))
        return rows16(rows_ref, off, _CH0 - _CH0 % 16, c, _CH0 % 16)

    def accum_b(rows_ref, e, c):
        off = e * _SEQ
        a0, a1, a2, a3 = lax.fori_loop(
            0, _CH1 // 16,
            lambda g, c: rows16(rows_ref, off, _CH0 + 16 * g, c, 16), c)
        out_v[e, pl.ds(0, 16)] = a0
        out_v[e, pl.ds(16, 16)] = a1
        out_v[e, pl.ds(32, 16)] = a2
        out_v[e, pl.ds(48, 16)] = a3

    slots = ((kidx0, rows0, (sem0a, sem0b)), (kidx1, rows1, (sem1a, sem1b)),
             (kidx2, rows2, (sem2a, sem2b)))
    fire(0, *slots[0])
    fire(1, *slots[1])
    fire(2, *slots[2])

    def step(k, _):
        for r, (kidx, rows_ref, sem) in enumerate(slots):
            e = 3 * k + r
            wait_a(rows_ref, sem)
            c = accum_a(rows_ref, e)
            wait_b(rows_ref, sem)
            accum_b(rows_ref, e, c)

            @pl.when(e + 3 < _BPW)
            def _():
                fire(e + 3, kidx, rows_ref, sem)
        return 0

    lax.fori_loop(0, _BPW // 3, step, 0)
    for r in range(_BPW % 3):
        e = (_BPW // 3) * 3 + r
        _, rows_ref, sem = slots[r]
        wait_a(rows_ref, sem)
        c = accum_a(rows_ref, e)
        wait_b(rows_ref, sem)
        accum_b(rows_ref, e, c)

    pltpu.sync_copy(out_v, out_hbm.at[pl.ds(wid * _BPW, _BPW)])


_pool = pl.kernel(
    _pool_body,
    out_type=jax.ShapeDtypeStruct((_BATCH, _EMBED), jnp.float32),
    mesh=_MESH,
    compiler_params=_SC_PARAMS,
    scratch_types=[
        pltpu.VMEM((_IDXW + 16,), jnp.int32),
        pltpu.VMEM((_SEQ,), jnp.int32),
        pltpu.VMEM((_SEQ,), jnp.int32),
        pltpu.VMEM((_SEQ,), jnp.int32),
        pltpu.VMEM((_SEQ, 2 * _EMBED), jnp.float32),
        pltpu.VMEM((_SEQ, 2 * _EMBED), jnp.float32),
        pltpu.VMEM((_SEQ, 2 * _EMBED), jnp.float32),
        pltpu.VMEM((_BPW, _EMBED), jnp.float32),
        pltpu.SemaphoreType.DMA,
        pltpu.SemaphoreType.DMA,
        pltpu.SemaphoreType.DMA,
        pltpu.SemaphoreType.DMA,
        pltpu.SemaphoreType.DMA,
        pltpu.SemaphoreType.DMA,
    ],
)


def _mlp_body(p_ref, w_ref, b_ref, o_ref):
    pooled = p_ref[...] * (1.0 / _SEQ)
    o_ref[...] = (
        jnp.dot(pooled, w_ref[...], preferred_element_type=jnp.float32)
        + b_ref[...])


_mlp = pl.pallas_call(
    _mlp_body,
    out_shape=jax.ShapeDtypeStruct((_BATCH, _OUT), jnp.float32),
)


@jax.jit
def kernel(x, table, W1, b1):
    x_flat = x.reshape(-1).astype(jnp.int32)
    pairs = _tconv(table.T)
    sums = _pool(x_flat, pairs)
    return _mlp(sums, W1, b1.reshape(1, _OUT))
